# baseline (speedup 1.0000x reference)
import jax
import jax.numpy as jnp
from jax.experimental import pallas as pl
from jax.experimental.pallas import tpu as pltpu

_NEG = float("-inf")
_BIGI = 1 << 22


def _nms_topk_body(x_ref, out_ref, nms_ref):
    xrow = jax.lax.broadcasted_iota(jnp.int32, (128, 1), 0)
    lf_iota = (jax.lax.broadcasted_iota(jnp.int32, (1, 128, 64), 1) * 64
               + jax.lax.broadcasted_iota(jnp.int32, (1, 128, 64), 2))
    lane16 = jax.lax.broadcasted_iota(jnp.int32, (1, 16), 1)
    x = x_ref[0]

    zm1 = jnp.concatenate([jnp.full((128, 128, 1), _NEG, jnp.float32), x[:, :, :-1]], axis=2)
    zp1 = jnp.concatenate([x[:, :, 1:], jnp.full((128, 128, 1), _NEG, jnp.float32)], axis=2)
    a = jnp.maximum(jnp.maximum(zm1, zp1), x)

    ym1 = jnp.concatenate([jnp.full((128, 1, 64), _NEG, jnp.float32), a[:, :-1, :]], axis=1)
    yp1 = jnp.concatenate([a[:, 1:, :], jnp.full((128, 1, 64), _NEG, jnp.float32)], axis=1)
    b = jnp.maximum(jnp.maximum(ym1, yp1), a)

    xm1 = jnp.concatenate([jnp.full((1, 128, 64), _NEG, jnp.float32), b[:-1, :, :]], axis=0)
    xp1 = jnp.concatenate([b[1:, :, :], jnp.full((1, 128, 64), _NEG, jnp.float32)], axis=0)
    m = jnp.maximum(jnp.maximum(xm1, xp1), b)

    nms = jnp.where(x == m, x, 0.0)
    nms_ref[...] = nms

    rmax = jnp.max(nms, axis=1)

    pickv = jnp.zeros((1, 16), jnp.float32)
    pickf = jnp.zeros((1, 16), jnp.int32)

    for k in range(10):
        gv = jnp.max(rmax)
        xs = jnp.min(jnp.where(rmax == gv, xrow, _BIGI))
        slab = nms_ref[pl.ds(xs, 1), :, :]
        lf = jnp.min(jnp.where(slab == gv, lf_iota, _BIGI))
        pickv = jnp.where(lane16 == k, gv, pickv)
        pickf = jnp.where(lane16 == k, xs * 8192 + lf, pickf)
        slab = jnp.where(lf_iota == lf, -1.0, slab)
        nms_ref[pl.ds(xs, 1), :, :] = slab
        newrow = jnp.max(slab, axis=1)
        rmax = jnp.where(xrow == xs, newrow, rmax)

    ixf = (pickf // 8192).astype(jnp.float32)
    iyf = ((pickf // 64) % 128).astype(jnp.float32)
    izf = (pickf % 64).astype(jnp.float32)
    locx = ixf / 127.0 * 8000.0 + 0.0 - 4000.0
    locy = iyf / 127.0 * 8000.0 + 0.0 - 4000.0
    locz = izf / 63.0 * 2000.0 + 800.0 - 1000.0
    zero = jnp.zeros((1, 16), jnp.float32)
    out_ref[0, 0:1, :] = locx
    out_ref[0, 1:2, :] = locy
    out_ref[0, 2:3, :] = locz
    out_ref[0, 3:4, :] = pickv
    out_ref[0, 4:5, :] = zero
    out_ref[0, 5:6, :] = zero
    out_ref[0, 6:7, :] = zero
    out_ref[0, 7:8, :] = zero


def kernel(root_cubes):
    out = pl.pallas_call(
        _nms_topk_body,
        grid=(8,),
        in_specs=[pl.BlockSpec((1, 128, 128, 64), lambda b: (b, 0, 0, 0))],
        out_specs=pl.BlockSpec((1, 8, 16), lambda b: (b, 0, 0)),
        out_shape=jax.ShapeDtypeStruct((8, 8, 16), jnp.float32),
        scratch_shapes=[pltpu.VMEM((128, 128, 64), jnp.float32)],
        compiler_params=pltpu.CompilerParams(
            vmem_limit_bytes=100 * 1024 * 1024,
        ),
    )(root_cubes)
    loc = jnp.stack([out[:, 0, :10], out[:, 1, :10], out[:, 2, :10]], axis=2)
    grid_centers = jnp.zeros((8, 10, 5), jnp.float32)
    grid_centers = grid_centers.at[:, :, 0:3].set(loc)
    grid_centers = grid_centers.at[:, :, 4].set(out[:, 3, :10])
    return grid_centers


# speedup vs baseline: 1.0187x; 1.0187x over previous
import jax
import jax.numpy as jnp
from jax.experimental import pallas as pl
from jax.experimental.pallas import tpu as pltpu

_NEG = float("-inf")
_BIGI = 1 << 22


def _nms_topk_body(x_ref, out_ref, nms_ref):
    xrow = jax.lax.broadcasted_iota(jnp.int32, (128, 1), 0)
    lf_iota = (jax.lax.broadcasted_iota(jnp.int32, (1, 128, 64), 1) * 64
               + jax.lax.broadcasted_iota(jnp.int32, (1, 128, 64), 2))
    lane16 = jax.lax.broadcasted_iota(jnp.int32, (1, 16), 1)
    x = x_ref[...]

    zm1 = jnp.concatenate([jnp.full((128, 128, 1), _NEG, jnp.float32), x[:, :, :-1]], axis=2)
    zp1 = jnp.concatenate([x[:, :, 1:], jnp.full((128, 128, 1), _NEG, jnp.float32)], axis=2)
    a = jnp.maximum(jnp.maximum(zm1, zp1), x)

    ym1 = jnp.concatenate([jnp.full((128, 1, 64), _NEG, jnp.float32), a[:, :-1, :]], axis=1)
    yp1 = jnp.concatenate([a[:, 1:, :], jnp.full((128, 1, 64), _NEG, jnp.float32)], axis=1)
    b = jnp.maximum(jnp.maximum(ym1, yp1), a)

    xm1 = jnp.concatenate([jnp.full((1, 128, 64), _NEG, jnp.float32), b[:-1, :, :]], axis=0)
    xp1 = jnp.concatenate([b[1:, :, :], jnp.full((1, 128, 64), _NEG, jnp.float32)], axis=0)
    m = jnp.maximum(jnp.maximum(xm1, xp1), b)

    nms = jnp.where(x == m, x, 0.0)
    nms_ref[...] = nms

    rmax = jnp.max(nms, axis=1)

    pickv = jnp.zeros((1, 16), jnp.float32)
    pickf = jnp.zeros((1, 16), jnp.int32)

    for k in range(10):
        gv = jnp.max(rmax)
        xs = jnp.min(jnp.where(rmax == gv, xrow, _BIGI))
        slab = nms_ref[pl.ds(xs, 1), :, :]
        lf = jnp.min(jnp.where(slab == gv, lf_iota, _BIGI))
        pickv = jnp.where(lane16 == k, gv, pickv)
        pickf = jnp.where(lane16 == k, xs * 8192 + lf, pickf)
        slab = jnp.where(lf_iota == lf, -1.0, slab)
        nms_ref[pl.ds(xs, 1), :, :] = slab
        newrow = jnp.max(slab, axis=1)
        rmax = jnp.where(xrow == xs, newrow, rmax)

    ixf = (pickf // 8192).astype(jnp.float32)
    iyf = ((pickf // 64) % 128).astype(jnp.float32)
    izf = (pickf % 64).astype(jnp.float32)
    locx = ixf / 127.0 * 8000.0 + 0.0 - 4000.0
    locy = iyf / 127.0 * 8000.0 + 0.0 - 4000.0
    locz = izf / 63.0 * 2000.0 + 800.0 - 1000.0
    zero = jnp.zeros((1, 16), jnp.float32)
    out_ref[0, 0:1, :] = locx
    out_ref[0, 1:2, :] = locy
    out_ref[0, 2:3, :] = locz
    out_ref[0, 3:4, :] = pickv
    out_ref[0, 4:5, :] = zero
    out_ref[0, 5:6, :] = zero
    out_ref[0, 6:7, :] = zero
    out_ref[0, 7:8, :] = zero


def kernel(root_cubes):
    x = root_cubes.reshape(1024, 128, 64)
    out = pl.pallas_call(
        _nms_topk_body,
        grid=(8,),
        in_specs=[pl.BlockSpec((128, 128, 64), lambda b: (b, 0, 0))],
        out_specs=pl.BlockSpec((1, 8, 16), lambda b: (b, 0, 0)),
        out_shape=jax.ShapeDtypeStruct((8, 8, 16), jnp.float32),
        scratch_shapes=[pltpu.VMEM((128, 128, 64), jnp.float32)],
        compiler_params=pltpu.CompilerParams(
            vmem_limit_bytes=100 * 1024 * 1024,
        ),
    )(x)
    loc = jnp.stack([out[:, 0, :10], out[:, 1, :10], out[:, 2, :10]], axis=2)
    grid_centers = jnp.zeros((8, 10, 5), jnp.float32)
    grid_centers = grid_centers.at[:, :, 0:3].set(loc)
    grid_centers = grid_centers.at[:, :, 4].set(out[:, 3, :10])
    return grid_centers
